# baseline (speedup 1.0000x reference)
import functools

import jax
import jax.numpy as jnp
from jax import lax
from jax.experimental import pallas as pl
from jax.experimental.pallas import tpu as pltpu
from jax.experimental.pallas import tpu_sc as plsc

_LANES = 128
_QROWS = 16
_TILES = 16
_CORES = 2


def _make_sc_agg(n, d, np_, ep, with_cnt):
    rows_per_tile = ep // (_CORES * _TILES) // _LANES
    slc = np_ // _TILES

    mesh = plsc.VectorSubcoreMesh(core_axis_name="c", subcore_axis_name="s")

    out_types = [jax.ShapeDtypeStruct((_CORES * np_, d), jnp.float32)]
    half = rows_per_tile // 2
    scratch = [
        pltpu.VMEM((half, _LANES), jnp.int32),
        pltpu.VMEM((half, _LANES), jnp.int32),
        pltpu.VMEM((2, _LANES, d), jnp.float32),
        pltpu.VMEM_SHARED((np_, d), jnp.float32),
        pltpu.SemaphoreType.DMA,
        pltpu.SemaphoreType.DMA,
    ]
    if with_cnt:
        out_types.append(jax.ShapeDtypeStruct((_CORES * np_,), jnp.float32))
        scratch += [
            pltpu.VMEM((_LANES,), jnp.float32),
            pltpu.VMEM_SHARED((np_,), jnp.float32),
            pltpu.VMEM((slc,), jnp.float32),
        ]

    def body(h, src2, dst2, zer2, zer1, ones1, p_out, cnt_out, sidx, didx,
             rows, acc, sem, sem_s, ones_v, cntacc, cz):
        c = lax.axis_index("c")
        s = lax.axis_index("s")
        base = s * slc
        pltpu.sync_copy(zer2.at[pl.ds(base, slc)], acc.at[pl.ds(base, slc)])
        if with_cnt:
            pltpu.sync_copy(zer1.at[pl.ds(base, slc)], cz)
            pltpu.sync_copy(cz, cntacc.at[pl.ds(base, slc)])
            pltpu.sync_copy(ones1, ones_v)
        wrow = (c * _TILES + s) * rows_per_tile
        plsc.subcore_barrier()

        for hh in range(2):
            hrow = wrow + hh * half
            pltpu.sync_copy(src2.at[pl.ds(hrow, half)], sidx)
            pltpu.sync_copy(dst2.at[pl.ds(hrow, half)], didx)
            pltpu.async_copy(h.at[sidx.at[0]], rows.at[0], sem)

            def jb(j, carry2):
                par = lax.rem(j, 2)
                nxt = lax.rem(j + 1, 2)
                pltpu.make_async_copy(h.at[sidx.at[j]], rows.at[par],
                                      sem).wait()
                pltpu.async_copy(rows.at[par], acc.at[didx.at[j]], sem_s,
                                 add=True)
                if with_cnt:
                    pltpu.sync_copy(ones_v, cntacc.at[didx.at[j]], add=True)

                @pl.when(j + 1 < half)
                def _():
                    @pl.when(j >= 1)
                    def _():
                        pltpu.make_async_copy(
                            rows.at[nxt],
                            acc.at[didx.at[lax.max(j - 1, 0)]],
                            sem_s).wait()
                    pltpu.async_copy(h.at[sidx.at[j + 1]], rows.at[nxt], sem)
                return carry2

            lax.fori_loop(0, half, jb, 0)
            for k in (half - 2, half - 1):
                pltpu.make_async_copy(rows.at[k % 2], acc.at[didx.at[k]],
                                      sem_s).wait()
        plsc.subcore_barrier()

        pltpu.sync_copy(acc.at[pl.ds(base, slc)],
                        p_out.at[pl.ds(c * np_ + base, slc)])
        if with_cnt:
            pltpu.sync_copy(cntacc.at[pl.ds(base, slc)], cz)
            pltpu.sync_copy(cz, cnt_out.at[pl.ds(c * np_ + base, slc)])

    if with_cnt:
        fn = body
    else:
        def fn(h, src2, dst2, zer2, p_out, sidx, didx, rows, acc, sem, sem_s):
            return body(h, src2, dst2, zer2, None, None, p_out, None, sidx,
                        didx, rows, acc, sem, sem_s, None, None, None)

    return pl.kernel(fn, mesh=mesh, out_type=out_types, scratch_types=scratch)


def _layer1_body(h_ref, p_ref, cnt_ref, wl_ref, bl_ref, wr_ref, g_ref, b_ref,
                 out_ref, inv_ref, *, n, np_):
    cnt = jnp.maximum(cnt_ref[0:n] + cnt_ref[np_:np_ + n], 1.0)
    inv = 1.0 / cnt
    inv_ref[...] = inv
    agg = p_ref[0:n] + p_ref[np_:np_ + n]
    mean = agg * inv
    y = jnp.dot(mean, wl_ref[...], preferred_element_type=jnp.float32)
    y = y + bl_ref[...]
    y = y + jnp.dot(h_ref[...], wr_ref[...], preferred_element_type=jnp.float32)
    mu = jnp.mean(y, axis=0, keepdims=True)
    var = jnp.mean((y - mu) * (y - mu), axis=0, keepdims=True)
    y = (y - mu) * lax.rsqrt(var + 1e-5) * g_ref[...] + b_ref[...]
    out_ref[...] = jnp.maximum(y, 0.0)


def _layer_body(h_ref, p_ref, inv_ref, wl_ref, bl_ref, wr_ref, g_ref, b_ref,
                out_ref, *, n, np_, final, wo_ref=None, bo_ref=None):
    agg = p_ref[0:n] + p_ref[np_:np_ + n]
    mean = agg * inv_ref[...]
    y = jnp.dot(mean, wl_ref[...], preferred_element_type=jnp.float32)
    y = y + bl_ref[...]
    y = y + jnp.dot(h_ref[...], wr_ref[...], preferred_element_type=jnp.float32)
    mu = jnp.mean(y, axis=0, keepdims=True)
    var = jnp.mean((y - mu) * (y - mu), axis=0, keepdims=True)
    y = (y - mu) * lax.rsqrt(var + 1e-5) * g_ref[...] + b_ref[...]
    h = jnp.maximum(y, 0.0)
    if final:
        out_ref[...] = jnp.dot(h, wo_ref[...],
                               preferred_element_type=jnp.float32) + bo_ref[...]
    else:
        out_ref[...] = h


def _mid_layer_call(h, p, inv, wl, bl, wr, g, b, n, d, np_):
    def body(h_ref, p_ref, inv_ref, wl_ref, bl_ref, wr_ref, g_ref, b_ref,
             out_ref):
        _layer_body(h_ref, p_ref, inv_ref, wl_ref, bl_ref, wr_ref, g_ref,
                    b_ref, out_ref, n=n, np_=np_, final=False)
    return pl.pallas_call(
        body, out_shape=jax.ShapeDtypeStruct((n, d), jnp.float32),
    )(h, p, inv, wl, bl.reshape(1, d), wr, g.reshape(1, d), b.reshape(1, d))


def _final_layer_call(h, p, inv, wl, bl, wr, g, b, wo, bo, n, d, np_):
    def body(h_ref, p_ref, inv_ref, wl_ref, bl_ref, wr_ref, g_ref, b_ref,
             wo_ref, bo_ref, out_ref):
        _layer_body(h_ref, p_ref, inv_ref, wl_ref, bl_ref, wr_ref, g_ref,
                    b_ref, out_ref, n=n, np_=np_, final=True, wo_ref=wo_ref,
                    bo_ref=bo_ref)
    return pl.pallas_call(
        body, out_shape=jax.ShapeDtypeStruct((n, d), jnp.float32),
    )(h, p, inv, wl, bl.reshape(1, d), wr, g.reshape(1, d), b.reshape(1, d),
      wo, bo.reshape(1, d))


def _layer1_call(h, p, cnt2d, wl, bl, wr, g, b, n, d, np_):
    def body(h_ref, p_ref, cnt_ref, wl_ref, bl_ref, wr_ref, g_ref, b_ref,
             out_ref, inv_ref):
        _layer1_body(h_ref, p_ref, cnt_ref, wl_ref, bl_ref, wr_ref, g_ref,
                     b_ref, out_ref, inv_ref, n=n, np_=np_)
    return pl.pallas_call(
        body,
        out_shape=[jax.ShapeDtypeStruct((n, d), jnp.float32),
                   jax.ShapeDtypeStruct((n, 1), jnp.float32)],
    )(h, p, cnt2d, wl, bl.reshape(1, d), wr, g.reshape(1, d), b.reshape(1, d))


def kernel(x, edge_index, Wl, bl, Wr, gamma, beta, Wo, bo):
    src = edge_index[0]
    dst = edge_index[1]
    n, d = x.shape
    e = src.shape[0]
    L = Wl.shape[0]

    np_ = ((n + _LANES - 1) // _LANES) * _LANES
    chunk_edges = _CORES * _TILES * _LANES * _QROWS
    ep = ((e + chunk_edges - 1) // chunk_edges) * chunk_edges
    pad_e = ep - e
    if pad_e:
        if np_ == n:
            np_ += _LANES
        ar = jnp.arange(pad_e, dtype=jnp.int32)
        srcp = jnp.concatenate([src, (ar * 997) % n])
        dstp = jnp.concatenate([dst, n + ar % (np_ - n)])
    else:
        srcp, dstp = src, dst
    src2 = srcp.reshape(ep // _LANES, _LANES)
    dst2 = dstp.reshape(ep // _LANES, _LANES)
    zer2 = jnp.zeros((np_, d), jnp.float32)
    zer1 = jnp.zeros((np_,), jnp.float32)
    ones1 = jnp.ones((_LANES,), jnp.float32)

    agg_first = _make_sc_agg(n, d, np_, ep, with_cnt=True)
    agg_rest = _make_sc_agg(n, d, np_, ep, with_cnt=False)

    h = x
    inv = None
    for i in range(L):
        if i == 0:
            p, cntp = agg_first(h, src2, dst2, zer2, zer1, ones1)
            h, inv = _layer1_call(h, p, cntp.reshape(_CORES * np_, 1), Wl[i],
                                  bl[i], Wr[i], gamma[i], beta[i], n, d, np_)
        elif i == L - 1:
            (p,) = agg_rest(h, src2, dst2, zer2)
            h = _final_layer_call(h, p, inv, Wl[i], bl[i], Wr[i], gamma[i],
                                  beta[i], Wo, bo, n, d, np_)
        else:
            (p,) = agg_rest(h, src2, dst2, zer2)
            h = _mid_layer_call(h, p, inv, Wl[i], bl[i], Wr[i], gamma[i],
                                beta[i], n, d, np_)
    return h


# speedup vs baseline: 1.1865x; 1.1865x over previous
import functools

import jax
import jax.numpy as jnp
from jax import lax
from jax.experimental import pallas as pl
from jax.experimental.pallas import tpu as pltpu
from jax.experimental.pallas import tpu_sc as plsc

_LANES = 128
_QROWS = 16
_TILES = 16
_CORES = 2


def _make_sc_agg(n, d, np_, ep, with_cnt):
    rows_per_tile = ep // (_CORES * _TILES) // _LANES
    slc = np_ // _TILES

    mesh = plsc.VectorSubcoreMesh(core_axis_name="c", subcore_axis_name="s")

    out_types = [jax.ShapeDtypeStruct((_CORES * np_, d), jnp.float32)]
    half = rows_per_tile // 2
    scratch = [
        pltpu.VMEM((half, _LANES), jnp.int32),
        pltpu.VMEM((half, _LANES), jnp.int32),
        pltpu.VMEM((2, _LANES, d), jnp.float32),
        pltpu.VMEM_SHARED((np_, d), jnp.float32),
        pltpu.SemaphoreType.DMA,
        pltpu.SemaphoreType.DMA,
    ]
    if with_cnt:
        out_types.append(jax.ShapeDtypeStruct((_CORES * np_,), jnp.float32))
        scratch += [
            pltpu.VMEM((_LANES,), jnp.float32),
            pltpu.VMEM_SHARED((np_,), jnp.float32),
            pltpu.VMEM((slc,), jnp.float32),
        ]

    def body(h, src2, dst2, zer2, zer1, ones1, p_out, cnt_out, sidx, didx,
             rows, acc, sem, sem_s, ones_v, cntacc, cz):
        c = lax.axis_index("c")
        s = lax.axis_index("s")
        base = s * slc
        pltpu.sync_copy(zer2.at[pl.ds(base, slc)], acc.at[pl.ds(base, slc)])
        if with_cnt:
            pltpu.sync_copy(zer1.at[pl.ds(base, slc)], cz)
            pltpu.sync_copy(cz, cntacc.at[pl.ds(base, slc)])
            pltpu.sync_copy(ones1, ones_v)
        wrow = (c * _TILES + s) * rows_per_tile
        plsc.subcore_barrier()

        for hh in range(2):
            hrow = wrow + hh * half
            pltpu.sync_copy(src2.at[pl.ds(hrow, half)], sidx)
            pltpu.sync_copy(dst2.at[pl.ds(hrow, half)], didx)
            pltpu.async_copy(h.at[sidx.at[0]], rows.at[0], sem)

            def jb(j, carry2):
                par = lax.rem(j, 2)
                nxt = lax.rem(j + 1, 2)

                @pl.when(j + 1 < half)
                def _():
                    pltpu.async_copy(h.at[sidx.at[j + 1]], rows.at[nxt], sem)

                pltpu.make_async_copy(h.at[sidx.at[j]], rows.at[par],
                                      sem).wait()
                pltpu.sync_copy(rows.at[par], acc.at[didx.at[j]], add=True)
                if with_cnt:
                    pltpu.sync_copy(ones_v, cntacc.at[didx.at[j]], add=True)
                return carry2

            lax.fori_loop(0, half, jb, 0)
        plsc.subcore_barrier()

        pltpu.sync_copy(acc.at[pl.ds(base, slc)],
                        p_out.at[pl.ds(c * np_ + base, slc)])
        if with_cnt:
            pltpu.sync_copy(cntacc.at[pl.ds(base, slc)], cz)
            pltpu.sync_copy(cz, cnt_out.at[pl.ds(c * np_ + base, slc)])

    if with_cnt:
        fn = body
    else:
        def fn(h, src2, dst2, zer2, p_out, sidx, didx, rows, acc, sem, sem_s):
            return body(h, src2, dst2, zer2, None, None, p_out, None, sidx,
                        didx, rows, acc, sem, sem_s, None, None, None)

    return pl.kernel(fn, mesh=mesh, out_type=out_types, scratch_types=scratch)


def _layer1_body(h_ref, p_ref, cnt_ref, wl_ref, bl_ref, wr_ref, g_ref, b_ref,
                 out_ref, inv_ref, *, n, np_):
    cnt = jnp.maximum(cnt_ref[0:n] + cnt_ref[np_:np_ + n], 1.0)
    inv = 1.0 / cnt
    inv_ref[...] = inv
    agg = p_ref[0:n] + p_ref[np_:np_ + n]
    mean = agg * inv
    y = jnp.dot(mean, wl_ref[...], preferred_element_type=jnp.float32)
    y = y + bl_ref[...]
    y = y + jnp.dot(h_ref[...], wr_ref[...], preferred_element_type=jnp.float32)
    mu = jnp.mean(y, axis=0, keepdims=True)
    var = jnp.mean((y - mu) * (y - mu), axis=0, keepdims=True)
    y = (y - mu) * lax.rsqrt(var + 1e-5) * g_ref[...] + b_ref[...]
    out_ref[...] = jnp.maximum(y, 0.0)


def _layer_body(h_ref, p_ref, inv_ref, wl_ref, bl_ref, wr_ref, g_ref, b_ref,
                out_ref, *, n, np_, final, wo_ref=None, bo_ref=None):
    agg = p_ref[0:n] + p_ref[np_:np_ + n]
    mean = agg * inv_ref[...]
    y = jnp.dot(mean, wl_ref[...], preferred_element_type=jnp.float32)
    y = y + bl_ref[...]
    y = y + jnp.dot(h_ref[...], wr_ref[...], preferred_element_type=jnp.float32)
    mu = jnp.mean(y, axis=0, keepdims=True)
    var = jnp.mean((y - mu) * (y - mu), axis=0, keepdims=True)
    y = (y - mu) * lax.rsqrt(var + 1e-5) * g_ref[...] + b_ref[...]
    h = jnp.maximum(y, 0.0)
    if final:
        out_ref[...] = jnp.dot(h, wo_ref[...],
                               preferred_element_type=jnp.float32) + bo_ref[...]
    else:
        out_ref[...] = h


def _mid_layer_call(h, p, inv, wl, bl, wr, g, b, n, d, np_):
    def body(h_ref, p_ref, inv_ref, wl_ref, bl_ref, wr_ref, g_ref, b_ref,
             out_ref):
        _layer_body(h_ref, p_ref, inv_ref, wl_ref, bl_ref, wr_ref, g_ref,
                    b_ref, out_ref, n=n, np_=np_, final=False)
    return pl.pallas_call(
        body, out_shape=jax.ShapeDtypeStruct((n, d), jnp.float32),
    )(h, p, inv, wl, bl.reshape(1, d), wr, g.reshape(1, d), b.reshape(1, d))


def _final_layer_call(h, p, inv, wl, bl, wr, g, b, wo, bo, n, d, np_):
    def body(h_ref, p_ref, inv_ref, wl_ref, bl_ref, wr_ref, g_ref, b_ref,
             wo_ref, bo_ref, out_ref):
        _layer_body(h_ref, p_ref, inv_ref, wl_ref, bl_ref, wr_ref, g_ref,
                    b_ref, out_ref, n=n, np_=np_, final=True, wo_ref=wo_ref,
                    bo_ref=bo_ref)
    return pl.pallas_call(
        body, out_shape=jax.ShapeDtypeStruct((n, d), jnp.float32),
    )(h, p, inv, wl, bl.reshape(1, d), wr, g.reshape(1, d), b.reshape(1, d),
      wo, bo.reshape(1, d))


def _layer1_call(h, p, cnt2d, wl, bl, wr, g, b, n, d, np_):
    def body(h_ref, p_ref, cnt_ref, wl_ref, bl_ref, wr_ref, g_ref, b_ref,
             out_ref, inv_ref):
        _layer1_body(h_ref, p_ref, cnt_ref, wl_ref, bl_ref, wr_ref, g_ref,
                     b_ref, out_ref, inv_ref, n=n, np_=np_)
    return pl.pallas_call(
        body,
        out_shape=[jax.ShapeDtypeStruct((n, d), jnp.float32),
                   jax.ShapeDtypeStruct((n, 1), jnp.float32)],
    )(h, p, cnt2d, wl, bl.reshape(1, d), wr, g.reshape(1, d), b.reshape(1, d))


def kernel(x, edge_index, Wl, bl, Wr, gamma, beta, Wo, bo):
    src = edge_index[0]
    dst = edge_index[1]
    n, d = x.shape
    e = src.shape[0]
    L = Wl.shape[0]

    np_ = ((n + _LANES - 1) // _LANES) * _LANES
    chunk_edges = _CORES * _TILES * _LANES * _QROWS
    ep = ((e + chunk_edges - 1) // chunk_edges) * chunk_edges
    pad_e = ep - e
    if pad_e:
        if np_ == n:
            np_ += _LANES
        ar = jnp.arange(pad_e, dtype=jnp.int32)
        srcp = jnp.concatenate([src, (ar * 997) % n])
        dstp = jnp.concatenate([dst, n + ar % (np_ - n)])
    else:
        srcp, dstp = src, dst
    src2 = srcp.reshape(ep // _LANES, _LANES)
    dst2 = dstp.reshape(ep // _LANES, _LANES)
    zer2 = jnp.zeros((np_, d), jnp.float32)
    zer1 = jnp.zeros((np_,), jnp.float32)
    ones1 = jnp.ones((_LANES,), jnp.float32)

    agg_first = _make_sc_agg(n, d, np_, ep, with_cnt=True)
    agg_rest = _make_sc_agg(n, d, np_, ep, with_cnt=False)

    h = x
    inv = None
    for i in range(L):
        if i == 0:
            p, cntp = agg_first(h, src2, dst2, zer2, zer1, ones1)
            h, inv = _layer1_call(h, p, cntp.reshape(_CORES * np_, 1), Wl[i],
                                  bl[i], Wr[i], gamma[i], beta[i], n, d, np_)
        elif i == L - 1:
            (p,) = agg_rest(h, src2, dst2, zer2)
            h = _final_layer_call(h, p, inv, Wl[i], bl[i], Wr[i], gamma[i],
                                  beta[i], Wo, bo, n, d, np_)
        else:
            (p,) = agg_rest(h, src2, dst2, zer2)
            h = _mid_layer_call(h, p, inv, Wl[i], bl[i], Wr[i], gamma[i],
                                beta[i], n, d, np_)
    return h
